# baseline (speedup 1.0000x reference)
import functools

import jax
import jax.numpy as jnp
from jax import lax
from jax.experimental import pallas as pl
from jax.experimental.pallas import tpu as pltpu
from jax.experimental.pallas import tpu_sc as plsc

BATCH = 1024
SEQ = 200
EMB = 64
NLANE = 16
NW = 32
PER_W = BATCH * SEQ // NW
C = 40
NCHUNK = PER_W // C
CPS = SEQ // C


def _sc_embed(idx2d, table, pos):
    mesh = plsc.VectorSubcoreMesh(core_axis_name="c", subcore_axis_name="s")

    @functools.partial(
        pl.kernel,
        mesh=mesh,
        out_type=jax.ShapeDtypeStruct((BATCH * SEQ, EMB), jnp.float32),
        compiler_params=pltpu.CompilerParams(use_tc_tiling_on_sc=False),
        scratch_types=[
            pltpu.VMEM((NCHUNK, C), jnp.int32),
            pltpu.VMEM((C, EMB), jnp.float32),
            pltpu.VMEM((SEQ, EMB), jnp.float32),
            pltpu.SemaphoreType.DMA,
        ],
    )
    def k(idx_hbm, table_hbm, pos_hbm, out_hbm, idx_v, rows_v, pos_v, sem):
        wid = lax.axis_index("s") * 2 + lax.axis_index("c")
        base = wid * PER_W
        pltpu.sync_copy(pos_hbm, pos_v)
        pltpu.sync_copy(idx_hbm.at[pl.ds(wid * NCHUNK, NCHUNK)], idx_v)

        def chunk_body(ci, carry):
            off = base + ci * C
            po = lax.rem(ci, CPS) * C
            pltpu.async_copy(table_hbm.at[idx_v.at[ci]], rows_v, sem).wait()

            def add_body(i, c2):
                for cc in range(EMB // NLANE):
                    s = pl.ds(cc * NLANE, NLANE)
                    rows_v[i, s] = rows_v[i, s] + pos_v[po + i, s]
                return c2

            lax.fori_loop(0, C, add_body, 0)
            pltpu.sync_copy(rows_v, out_hbm.at[pl.ds(off, C)])
            return carry

        lax.fori_loop(0, NCHUNK, chunk_body, 0)

    return k(idx2d, table, pos)


def kernel(x, tok_emb, pos_emb):
    idx2d = x.reshape(NW * NCHUNK, C)
    pos = pos_emb[0, :SEQ, :]
    out = _sc_embed(idx2d, tok_emb, pos)
    return out.reshape(BATCH, SEQ, EMB)


# speedup vs baseline: 1.2905x; 1.2905x over previous
import functools

import jax
import jax.numpy as jnp
from jax import lax
from jax.experimental import pallas as pl
from jax.experimental.pallas import tpu as pltpu
from jax.experimental.pallas import tpu_sc as plsc

BATCH = 1024
SEQ = 200
EMB = 64
NLANE = 16
NW = 32
PER_W = BATCH * SEQ // NW
C = 40
NCHUNK = PER_W // C
S = 2
BLK = S * SEQ
NB = PER_W // BLK
GPB = BLK // C


def _sc_embed(idx2d, table, pos):
    mesh = plsc.VectorSubcoreMesh(core_axis_name="c", subcore_axis_name="s")

    @functools.partial(
        pl.kernel,
        mesh=mesh,
        out_type=jax.ShapeDtypeStruct((BATCH * SEQ, EMB), jnp.float32),
        compiler_params=pltpu.CompilerParams(use_tc_tiling_on_sc=False),
        scratch_types=[
            pltpu.VMEM((NCHUNK, C), jnp.int32),
            pltpu.VMEM((BLK, EMB), jnp.float32),
            pltpu.VMEM((BLK, EMB), jnp.float32),
            pltpu.VMEM((SEQ, EMB), jnp.float32),
            pltpu.SemaphoreType.DMA,
            pltpu.SemaphoreType.DMA,
        ],
    )
    def k(idx_hbm, table_hbm, pos_hbm, out_hbm, idx_v, buf0, buf1, pos_v,
          gsem, osem):
        wid = lax.axis_index("s") * 2 + lax.axis_index("c")
        base = wid * PER_W
        pltpu.sync_copy(pos_hbm, pos_v)
        pltpu.sync_copy(idx_hbm.at[pl.ds(wid * NCHUNK, NCHUNK)], idx_v)

        bufs = (buf0, buf1)

        def fire_gathers(b, buf):
            return [
                pltpu.async_copy(
                    table_hbm.at[idx_v.at[b * GPB + j]],
                    buf.at[pl.ds(j * C, C)],
                    gsem,
                )
                for j in range(GPB)
            ]

        def add_pos(buf):
            def body(i, carry):
                for cc in range(EMB // NLANE):
                    sl = pl.ds(cc * NLANE, NLANE)
                    pv = pos_v[i, sl]
                    for s in range(S):
                        buf[s * SEQ + i, sl] = buf[s * SEQ + i, sl] + pv
                return carry

            lax.fori_loop(0, SEQ, body, 0)

        gd = {0: fire_gathers(0, bufs[0])}
        od = {}
        for b in range(NB):
            q = b % 2
            for d in gd.pop(b):
                d.wait()
            if b + 1 < NB:
                if b >= 1:
                    od.pop(b - 1).wait()
                gd[b + 1] = fire_gathers(b + 1, bufs[(b + 1) % 2])
            add_pos(bufs[q])
            od[b] = pltpu.async_copy(
                bufs[q], out_hbm.at[pl.ds(base + b * BLK, BLK)], osem)
        od.pop(NB - 2).wait()
        od.pop(NB - 1).wait()

    return k(idx2d, table, pos)


def kernel(x, tok_emb, pos_emb):
    idx2d = x.reshape(NW * NCHUNK, C)
    pos = pos_emb[0, :SEQ, :]
    out = _sc_embed(idx2d, tok_emb, pos)
    return out.reshape(BATCH, SEQ, EMB)


# speedup vs baseline: 1.4901x; 1.1547x over previous
import functools

import jax
import jax.numpy as jnp
from jax import lax
from jax.experimental import pallas as pl
from jax.experimental.pallas import tpu as pltpu
from jax.experimental.pallas import tpu_sc as plsc

BATCH = 1024
SEQ = 200
EMB = 64
PAD = 128
NLANE = 16
NW = 32
PER_W = BATCH * SEQ // NW
C = 40
NCHUNK = PER_W // C
BLK = SEQ
NB = PER_W // BLK
GPB = BLK // C


def _sc_embed(idx2d, table, pos):
    mesh = plsc.VectorSubcoreMesh(core_axis_name="c", subcore_axis_name="s")

    @functools.partial(
        pl.kernel,
        mesh=mesh,
        out_type=jax.ShapeDtypeStruct((BATCH * SEQ, PAD), jnp.float32),
        scratch_types=[
            pltpu.VMEM((NCHUNK, C), jnp.int32),
            pltpu.VMEM((BLK, PAD), jnp.float32),
            pltpu.VMEM((BLK, PAD), jnp.float32),
            pltpu.VMEM((SEQ, EMB), jnp.float32),
            pltpu.SemaphoreType.DMA,
            pltpu.SemaphoreType.DMA,
        ],
    )
    def k(idx_hbm, table_hbm, pos_hbm, out_hbm, idx_v, buf0, buf1, pos_v,
          gsem, osem):
        wid = lax.axis_index("s") * 2 + lax.axis_index("c")
        base = wid * PER_W
        pltpu.sync_copy(pos_hbm, pos_v)
        pltpu.sync_copy(idx_hbm.at[pl.ds(wid * NCHUNK, NCHUNK)], idx_v)

        bufs = (buf0, buf1)

        def fire_gathers(b, buf):
            return [
                pltpu.async_copy(
                    table_hbm.at[idx_v.at[b * GPB + j]],
                    buf.at[pl.ds(j * C, C)],
                    gsem,
                )
                for j in range(GPB)
            ]

        def add_pos(buf):
            def body(i, carry):
                for cc in range(EMB // NLANE):
                    sl = pl.ds(cc * NLANE, NLANE)
                    buf[i, sl] = buf[i, sl] + pos_v[i, sl]
                return carry

            lax.fori_loop(0, SEQ, body, 0)

        gd = {0: fire_gathers(0, bufs[0])}
        od = {}
        for b in range(NB):
            q = b % 2
            for d in gd.pop(b):
                d.wait()
            if b + 1 < NB:
                if b >= 1:
                    od.pop(b - 1).wait()
                gd[b + 1] = fire_gathers(b + 1, bufs[(b + 1) % 2])
            add_pos(bufs[q])
            od[b] = pltpu.async_copy(
                bufs[q], out_hbm.at[pl.ds(base + b * BLK, BLK)], osem)
        od.pop(NB - 2).wait()
        od.pop(NB - 1).wait()

    return k(idx2d, table, pos)


def kernel(x, tok_emb, pos_emb):
    idx2d = x.reshape(NW * NCHUNK, C)
    tok128 = jnp.pad(tok_emb, ((0, 0), (0, PAD - EMB)))
    pos = pos_emb[0, :SEQ, :]
    out = _sc_embed(idx2d, tok128, pos)
    return out[:, :EMB].reshape(BATCH, SEQ, EMB)
